# Initial kernel scaffold; baseline (speedup 1.0000x reference)
import jax
import jax.numpy as jnp
from jax.experimental import pallas as pl


def kernel(feat_n, feat_l, src_l2n, dst_l2n, src_n2l, dst_n2l, params):
    raise NotImplementedError("write your pallas kernel here")


# speedup vs baseline: 1.8367x; 1.8367x over previous
import functools

import jax
import jax.numpy as jnp
from jax import lax
from jax.experimental import pallas as pl
from jax.experimental.pallas import tpu as pltpu
from jax.experimental.pallas import tpu_sc as plsc

N_NODES = 50000
N_PAD = 50176
TRASH = 50100
CW = 32
DW = 16
ROWS_PER_TILE = N_PAD // 16
EB = 128

_MESH = dict(core_axis_name="c", subcore_axis_name="s",
             num_cores=2, num_subcores=16)


def _sc_degrees(a0, a1, a2, a3):
    e_pad = a0.shape[0]
    ept = e_pad // 16
    nchk = ept // EB
    half = ROWS_PER_TILE // 2

    out_t = [jax.ShapeDtypeStruct((N_PAD, DW), jnp.float32)] * 4
    scratch = [
        pltpu.VMEM_SHARED((N_PAD, DW), jnp.float32),
        pltpu.VMEM_SHARED((N_PAD, DW), jnp.float32),
        pltpu.VMEM((EB,), jnp.int32),
        pltpu.VMEM((EB, DW), jnp.float32),
        pltpu.VMEM((half, DW), jnp.float32),
    ]

    @functools.partial(
        pl.kernel,
        out_type=out_t,
        mesh=plsc.VectorSubcoreMesh(**_MESH),
        scratch_types=scratch,
        compiler_params=pltpu.CompilerParams(use_tc_tiling_on_sc=False),
    )
    def k(r0_, r1_, r2_, r3_, o0, o1, o2, o3, acc0, acc1, ibuf, ones, tmp):
        c = lax.axis_index("c")
        s = lax.axis_index("s")
        arrs = (r0_, r1_, r2_, r3_)
        outs = (o0, o1, o2, o3)
        accs = (acc0, acc1)
        one16 = jnp.ones((DW,), jnp.float32)
        zero16 = jnp.zeros((DW,), jnp.float32)

        @pl.loop(0, EB)
        def _(i):
            ones[i, :] = one16

        @pl.loop(0, half)
        def _(i):
            tmp[i, :] = zero16

        r0 = s * ROWS_PER_TILE
        for acc in accs:
            pltpu.sync_copy(tmp, acc.at[pl.ds(r0, half)])
            pltpu.sync_copy(tmp, acc.at[pl.ds(r0 + half, half)])
        plsc.subcore_barrier()

        base_e = s * ept
        for cc in range(2):

            @pl.when(c == cc)
            def _():
                for j in range(2):
                    arr = arrs[2 * cc + j]
                    acc = accs[j]

                    @pl.loop(0, nchk)
                    def _(i):
                        off = base_e + i * EB
                        pltpu.sync_copy(arr.at[pl.ds(off, EB)], ibuf)
                        pltpu.sync_copy(ones, acc.at[ibuf], add=True)

        plsc.subcore_barrier()
        for cc in range(2):

            @pl.when(c == cc)
            def _():
                for j in range(2):
                    acc = accs[j]
                    out = outs[2 * cc + j]
                    for q in range(2):
                        rr = r0 + q * half
                        pltpu.sync_copy(acc.at[pl.ds(rr, half)], tmp)
                        pltpu.sync_copy(tmp, out.at[pl.ds(rr, half)])

    return k(a0, a1, a2, a3)


def _sc_segsum(h0, h1, h2, h3, src, dst):
    e_pad = src.shape[0]
    ept = e_pad // 16
    nchk = ept // EB
    quarter = ROWS_PER_TILE // 4

    out_t = [jax.ShapeDtypeStruct((N_PAD, CW), jnp.float32)] * 4
    scratch = [
        pltpu.VMEM_SHARED((N_PAD, CW), jnp.float32),
        pltpu.VMEM((EB,), jnp.int32),
        pltpu.VMEM((EB,), jnp.int32),
        pltpu.VMEM((EB, CW), jnp.float32),
        pltpu.VMEM((quarter, CW), jnp.float32),
        pltpu.SemaphoreType.DMA,
    ]

    @functools.partial(
        pl.kernel,
        out_type=out_t,
        mesh=plsc.VectorSubcoreMesh(**_MESH),
        scratch_types=scratch,
        compiler_params=pltpu.CompilerParams(use_tc_tiling_on_sc=False),
    )
    def k(h0_, h1_, h2_, h3_, se, de, o0, o1, o2, o3, acc, sbuf, dbuf, rows,
          tmp, sem):
        c = lax.axis_index("c")
        s = lax.axis_index("s")
        hs = (h0_, h1_, h2_, h3_)
        outs = (o0, o1, o2, o3)
        z16 = jnp.zeros((16,), jnp.float32)
        r0 = s * ROWS_PER_TILE
        base_e = s * ept

        for cc in range(2):

            @pl.when(c == cc)
            def _():
                for p in range(2):
                    ch = 2 * cc + p

                    @pl.loop(0, quarter)
                    def _(i):
                        tmp[i, pl.ds(0, 16)] = z16
                        tmp[i, pl.ds(16, 16)] = z16

                    for q in range(4):
                        pltpu.sync_copy(tmp, acc.at[pl.ds(r0 + q * quarter,
                                                          quarter)])
                    plsc.subcore_barrier()

                    @pl.loop(0, nchk)
                    def _(i):
                        off = base_e + i * EB
                        pltpu.sync_copy(se.at[pl.ds(off, EB)], sbuf)
                        pltpu.sync_copy(de.at[pl.ds(off, EB)], dbuf)
                        pltpu.async_copy(hs[ch].at[sbuf], rows, sem).wait()
                        pltpu.sync_copy(rows, acc.at[dbuf], add=True)

                    plsc.subcore_barrier()
                    for q in range(4):
                        rr = r0 + q * quarter
                        pltpu.sync_copy(acc.at[pl.ds(rr, quarter)], tmp)
                        pltpu.sync_copy(tmp, outs[ch].at[pl.ds(rr, quarter)])
                    plsc.subcore_barrier()

    return k(h0, h1, h2, h3, src, dst)


_BN = 512
_GRID = N_PAD // _BN


def _tc_project(x, w, deg):
    kdim = x.shape[1]

    def body(x_ref, w_ref, d_ref, o0, o1, o2, o3):
        ns = lax.rsqrt(jnp.maximum(d_ref[:, 0:1], 1.0))
        h = jnp.dot(x_ref[...], w_ref[...],
                    preferred_element_type=jnp.float32) * ns
        for p, o in enumerate((o0, o1, o2, o3)):
            o[...] = h[:, p * CW:(p + 1) * CW]

    return pl.pallas_call(
        body,
        grid=(_GRID,),
        in_specs=[
            pl.BlockSpec((_BN, kdim), lambda i: (i, 0)),
            pl.BlockSpec((kdim, 128), lambda i: (0, 0)),
            pl.BlockSpec((_BN, DW), lambda i: (i, 0)),
        ],
        out_specs=[pl.BlockSpec((_BN, CW), lambda i: (i, 0))] * 4,
        out_shape=[jax.ShapeDtypeStruct((N_PAD, CW), jnp.float32)] * 4,
    )(x, w, deg)


def _postprocess(a_refs, di_ref, b_ref, g_ref, lb_ref):
    a = jnp.concatenate([r[...] for r in a_refs], axis=1)
    nd = lax.rsqrt(jnp.maximum(di_ref[:, 0:1], 1.0))
    x = a * nd + b_ref[...]
    mu = jnp.mean(x, axis=1, keepdims=True)
    xc = x - mu
    var = jnp.mean(xc * xc, axis=1, keepdims=True)
    xh = xc * lax.rsqrt(var + 1e-5) * g_ref[...] + lb_ref[...]
    return jnp.where(xh > 0, xh, jnp.exp(jnp.minimum(xh, 0.0)) - 1.0)


def _tc_post_mm(a0, a1, a2, a3, deg_in, b, g, lb, deg_out, w_next):
    def body(a0_, a1_, a2_, a3_, di, b_, g_, lb_, do, w_, o0, o1, o2, o3):
        h = _postprocess((a0_, a1_, a2_, a3_), di, b_, g_, lb_)
        ns = lax.rsqrt(jnp.maximum(do[:, 0:1], 1.0))
        o = jnp.dot(h * ns, w_[...], preferred_element_type=jnp.float32)
        for p, oo in enumerate((o0, o1, o2, o3)):
            oo[...] = o[:, p * CW:(p + 1) * CW]

    chunk = pl.BlockSpec((_BN, CW), lambda i: (i, 0))
    vec = pl.BlockSpec((1, 128), lambda i: (0, 0))
    return pl.pallas_call(
        body,
        grid=(_GRID,),
        in_specs=[chunk] * 4 + [
            pl.BlockSpec((_BN, DW), lambda i: (i, 0)), vec, vec, vec,
            pl.BlockSpec((_BN, DW), lambda i: (i, 0)),
            pl.BlockSpec((128, 128), lambda i: (0, 0)),
        ],
        out_specs=[chunk] * 4,
        out_shape=[jax.ShapeDtypeStruct((N_PAD, CW), jnp.float32)] * 4,
    )(a0, a1, a2, a3, deg_in, b, g, lb, deg_out, w_next)


def _tc_post_final(a0, a1, a2, a3, deg_in, b, g, lb):
    def body(a0_, a1_, a2_, a3_, di, b_, g_, lb_, o):
        h = _postprocess((a0_, a1_, a2_, a3_), di, b_, g_, lb_)
        row = (pl.program_id(0) * _BN
               + lax.broadcasted_iota(jnp.int32, (_BN, 1), 0))
        o[...] = jnp.where(row < N_NODES, h, 0.0)

    chunk = pl.BlockSpec((_BN, CW), lambda i: (i, 0))
    vec = pl.BlockSpec((1, 128), lambda i: (0, 0))
    return pl.pallas_call(
        body,
        grid=(_GRID,),
        in_specs=[chunk] * 4 + [
            pl.BlockSpec((_BN, DW), lambda i: (i, 0)), vec, vec, vec,
        ],
        out_specs=pl.BlockSpec((_BN, 128), lambda i: (i, 0)),
        out_shape=jax.ShapeDtypeStruct((N_PAD, 128), jnp.float32),
    )(a0, a1, a2, a3, deg_in, b, g, lb)


def _tc_colsum(h):
    def body(h_ref, o_ref):
        @pl.when(pl.program_id(0) == 0)
        def _():
            o_ref[...] = jnp.zeros_like(o_ref)

        part = jnp.sum(h_ref[...], axis=0, keepdims=True)
        o_ref[...] += jnp.broadcast_to(part, (8, 128))

    return pl.pallas_call(
        body,
        grid=(_GRID,),
        in_specs=[pl.BlockSpec((_BN, 128), lambda i: (i, 0))],
        out_specs=pl.BlockSpec((8, 128), lambda i: (0, 0)),
        out_shape=jax.ShapeDtypeStruct((8, 128), jnp.float32),
    )(h)


def _tc_head(sn, sl, w0, b0, w1, b1):
    def body(sn_, sl_, w0_, b0_, w1_, b1_, o):
        hg = (sn_[0:1, :] + sl_[0:1, :]) * (1.0 / N_NODES)
        t = jnp.maximum(
            jnp.dot(hg, w0_[...], preferred_element_type=jnp.float32)
            + b0_[...], 0.0)
        r = jnp.dot(t, w1_[...], preferred_element_type=jnp.float32) + b1_[...]
        o[...] = jnp.broadcast_to(r, (8, 128))

    return pl.pallas_call(
        body,
        out_shape=jax.ShapeDtypeStruct((8, 128), jnp.float32),
    )(sn, sl, w0, b0, w1, b1)


def kernel(feat_n, feat_l, src_l2n, dst_l2n, src_n2l, dst_n2l, params):
    f32 = jnp.float32
    i32 = jnp.int32

    e = src_l2n.shape[0]
    e_pad = ((e + 2047) // 2048) * 2048

    def pad_idx(a):
        return jnp.concatenate(
            [a.astype(i32), jnp.full((e_pad - e,), TRASH, i32)])

    sl2n, dl2n = pad_idx(src_l2n), pad_idx(dst_l2n)
    sn2l, dn2l = pad_idx(src_n2l), pad_idx(dst_n2l)

    kn = feat_n.shape[1]
    kl = feat_l.shape[1]
    kn_p = ((kn + 127) // 128) * 128
    kl_p = ((kl + 127) // 128) * 128
    fn = jnp.pad(feat_n.astype(f32), ((0, N_PAD - feat_n.shape[0]),
                                      (0, kn_p - kn)))
    fl = jnp.pad(feat_l.astype(f32), ((0, N_PAD - feat_l.shape[0]),
                                      (0, kl_p - kl)))
    w0_l2n = jnp.pad(params["W0_l2n"].astype(f32), ((0, kl_p - kl), (0, 0)))
    w0_n2l = jnp.pad(params["W0_n2l"].astype(f32), ((0, kn_p - kn), (0, 0)))

    d0, d1, d2, d3 = _sc_degrees(sl2n, dl2n, sn2l, dn2l)

    vec = lambda v: v.astype(f32).reshape(1, -1)

    h_l2n = _tc_project(fl, w0_l2n, d0)
    h_n2l = _tc_project(fn, w0_n2l, d2)

    for i in range(3):
        a_n = _sc_segsum(*h_l2n, sl2n, dl2n)
        a_l = _sc_segsum(*h_n2l, sn2l, dn2l)
        b_n = vec(params["b%d_l2n" % i])
        b_l = vec(params["b%d_n2l" % i])
        g_n, lb_n = vec(params["ln%d_n_g" % i]), vec(params["ln%d_n_b" % i])
        g_l, lb_l = vec(params["ln%d_l_g" % i]), vec(params["ln%d_l_b" % i])
        if i < 2:
            h_n2l = _tc_post_mm(*a_n, d1, b_n, g_n, lb_n, d2,
                                params["W%d_n2l" % (i + 1)].astype(f32))
            h_l2n = _tc_post_mm(*a_l, d3, b_l, g_l, lb_l, d0,
                                params["W%d_l2n" % (i + 1)].astype(f32))
        else:
            hn = _tc_post_final(*a_n, d1, b_n, g_n, lb_n)
            hl = _tc_post_final(*a_l, d3, b_l, g_l, lb_l)

    sn = _tc_colsum(hn)
    sl = _tc_colsum(hl)
    w1 = jnp.pad(params["out_W"].astype(f32), ((0, 0), (0, 127)))
    b1 = jnp.pad(params["out_b"].astype(f32).reshape(1, 1),
                 ((0, 0), (0, 127)))
    res = _tc_head(sn, sl, params["fc0_W"].astype(f32), vec(params["fc0_b"]),
                   w1, b1)
    return res[0:1, 0:1]


# speedup vs baseline: 3.8441x; 2.0930x over previous
import functools

import jax
import jax.numpy as jnp
from jax import lax
from jax.experimental import pallas as pl
from jax.experimental.pallas import tpu as pltpu
from jax.experimental.pallas import tpu_sc as plsc

N_NODES = 50000
N_PAD = 50176
TRASH = 50100
CW = 32
DW = 16
ROWS_PER_TILE = N_PAD // 16
EB = 128

_MESH = dict(core_axis_name="c", subcore_axis_name="s",
             num_cores=2, num_subcores=16)


def _sc_degrees(a0, a1, a2, a3):
    e_pad = a0.shape[0]
    ept = e_pad // 16
    nchk = ept // EB
    half = ROWS_PER_TILE // 2

    out_t = [jax.ShapeDtypeStruct((N_PAD, DW), jnp.float32)] * 4
    scratch = [
        pltpu.VMEM_SHARED((N_PAD, DW), jnp.float32),
        pltpu.VMEM_SHARED((N_PAD, DW), jnp.float32),
        pltpu.VMEM((EB,), jnp.int32),
        pltpu.VMEM((EB, DW), jnp.float32),
        pltpu.VMEM((half, DW), jnp.float32),
    ]

    @functools.partial(
        pl.kernel,
        out_type=out_t,
        mesh=plsc.VectorSubcoreMesh(**_MESH),
        scratch_types=scratch,
        compiler_params=pltpu.CompilerParams(use_tc_tiling_on_sc=False),
    )
    def k(r0_, r1_, r2_, r3_, o0, o1, o2, o3, acc0, acc1, ibuf, ones, tmp):
        c = lax.axis_index("c")
        s = lax.axis_index("s")
        arrs = (r0_, r1_, r2_, r3_)
        outs = (o0, o1, o2, o3)
        accs = (acc0, acc1)
        one16 = jnp.ones((DW,), jnp.float32)
        zero16 = jnp.zeros((DW,), jnp.float32)

        @pl.loop(0, EB)
        def _(i):
            ones[i, :] = one16

        @pl.loop(0, half)
        def _(i):
            tmp[i, :] = zero16

        r0 = s * ROWS_PER_TILE
        for acc in accs:
            pltpu.sync_copy(tmp, acc.at[pl.ds(r0, half)])
            pltpu.sync_copy(tmp, acc.at[pl.ds(r0 + half, half)])
        plsc.subcore_barrier()

        base_e = s * ept
        for cc in range(2):

            @pl.when(c == cc)
            def _():
                for j in range(2):
                    arr = arrs[2 * cc + j]
                    acc = accs[j]

                    @pl.loop(0, nchk)
                    def _(i):
                        off = base_e + i * EB
                        pltpu.sync_copy(arr.at[pl.ds(off, EB)], ibuf)
                        pltpu.sync_copy(ones, acc.at[ibuf], add=True)

        plsc.subcore_barrier()
        for cc in range(2):

            @pl.when(c == cc)
            def _():
                for j in range(2):
                    acc = accs[j]
                    out = outs[2 * cc + j]
                    for q in range(2):
                        rr = r0 + q * half
                        pltpu.sync_copy(acc.at[pl.ds(rr, half)], tmp)
                        pltpu.sync_copy(tmp, out.at[pl.ds(rr, half)])

    return k(a0, a1, a2, a3)


SBE = 256
STRIPS = SBE // EB
TQ = 196


def _sc_segsum(h0, h1, h2, h3, src2, dst2):
    erows = src2.shape[0]
    rpt = erows // 16
    sb = rpt // STRIPS
    out_t = [jax.ShapeDtypeStruct((N_PAD, CW), jnp.float32)] * 4
    scratch = [
        pltpu.VMEM_SHARED((N_PAD, CW), jnp.float32),
        pltpu.VMEM((3, STRIPS, EB), jnp.int32),
        pltpu.VMEM((3, STRIPS, EB), jnp.int32),
        pltpu.VMEM((2, SBE, CW), jnp.float32),
        pltpu.VMEM((TQ, CW), jnp.float32),
        pltpu.SemaphoreType.DMA,
        pltpu.SemaphoreType.DMA,
        pltpu.SemaphoreType.DMA,
    ]

    @functools.partial(
        pl.kernel,
        out_type=out_t,
        mesh=plsc.VectorSubcoreMesh(**_MESH),
        scratch_types=scratch,
        compiler_params=pltpu.CompilerParams(use_tc_tiling_on_sc=False),
    )
    def k(h0_, h1_, h2_, h3_, se, de, o0, o1, o2, o3, acc, sbuf, dbuf, rows,
          tmp, isem, gsem, ssem):
        c = lax.axis_index("c")
        s = lax.axis_index("s")
        hs = (h0_, h1_, h2_, h3_)
        outs = (o0, o1, o2, o3)
        z16 = jnp.zeros((16,), jnp.float32)
        r0 = s * ROWS_PER_TILE
        base_r = s * rpt

        def idx_dma(g, slot, wait):
            srcs = se.at[pl.ds(base_r + STRIPS * g, STRIPS)]
            dsts = de.at[pl.ds(base_r + STRIPS * g, STRIPS)]
            if wait:
                pltpu.make_async_copy(srcs, sbuf.at[slot], isem).wait()
                pltpu.make_async_copy(dsts, dbuf.at[slot], isem).wait()
            else:
                pltpu.async_copy(srcs, sbuf.at[slot], isem)
                pltpu.async_copy(dsts, dbuf.at[slot], isem)

        for cc in range(2):

            @pl.when(c == cc)
            def _():
                for p in range(2):
                    ch = 2 * cc + p
                    h = hs[ch]
                    drain_src = h.at[pl.ds(0, SBE)]

                    @pl.loop(0, TQ)
                    def _(i):
                        tmp[i, pl.ds(0, 16)] = z16
                        tmp[i, pl.ds(16, 16)] = z16

                    for q in range(16):
                        pltpu.sync_copy(tmp, acc.at[pl.ds(r0 + q * TQ, TQ)])
                    plsc.subcore_barrier()

                    idx_dma(0, 0, wait=False)

                    @pl.loop(0, sb)
                    def _(g):
                        slot = lax.rem(g, 3)
                        rb = lax.rem(g, 2)

                        @pl.when(g >= 1)
                        def _():
                            pltpu.make_async_copy(drain_src,
                                                  rows.at[1 - rb],
                                                  ssem).wait()

                        idx_dma(g, slot, wait=True)
                        for j in range(STRIPS):
                            pltpu.async_copy(
                                h.at[sbuf.at[slot, j]],
                                rows.at[rb, pl.ds(j * EB, EB)], gsem)

                        @pl.when(g + 1 < sb)
                        def _():
                            idx_dma(g + 1, lax.rem(g + 1, 3), wait=False)

                        pltpu.make_async_copy(drain_src, rows.at[rb],
                                              gsem).wait()
                        for j in range(STRIPS):
                            pltpu.async_copy(
                                rows.at[rb, pl.ds(j * EB, EB)],
                                acc.at[dbuf.at[slot, j]], ssem, add=True)

                    pltpu.make_async_copy(drain_src, rows.at[0],
                                          ssem).wait()
                    plsc.subcore_barrier()
                    for q in range(16):
                        rr = r0 + q * TQ
                        pltpu.sync_copy(acc.at[pl.ds(rr, TQ)], tmp)
                        pltpu.sync_copy(tmp, outs[ch].at[pl.ds(rr, TQ)])
                    plsc.subcore_barrier()

    return k(h0, h1, h2, h3, src2, dst2)


_BN = 512
_GRID = N_PAD // _BN


def _tc_project(x, w, deg):
    kdim = x.shape[1]

    def body(x_ref, w_ref, d_ref, o0, o1, o2, o3):
        ns = lax.rsqrt(jnp.maximum(d_ref[:, 0:1], 1.0))
        h = jnp.dot(x_ref[...], w_ref[...],
                    preferred_element_type=jnp.float32) * ns
        for p, o in enumerate((o0, o1, o2, o3)):
            o[...] = h[:, p * CW:(p + 1) * CW]

    return pl.pallas_call(
        body,
        grid=(_GRID,),
        in_specs=[
            pl.BlockSpec((_BN, kdim), lambda i: (i, 0)),
            pl.BlockSpec((kdim, 128), lambda i: (0, 0)),
            pl.BlockSpec((_BN, DW), lambda i: (i, 0)),
        ],
        out_specs=[pl.BlockSpec((_BN, CW), lambda i: (i, 0))] * 4,
        out_shape=[jax.ShapeDtypeStruct((N_PAD, CW), jnp.float32)] * 4,
    )(x, w, deg)


def _postprocess(a_refs, di_ref, b_ref, g_ref, lb_ref):
    a = jnp.concatenate([r[...] for r in a_refs], axis=1)
    nd = lax.rsqrt(jnp.maximum(di_ref[:, 0:1], 1.0))
    x = a * nd + b_ref[...]
    mu = jnp.mean(x, axis=1, keepdims=True)
    xc = x - mu
    var = jnp.mean(xc * xc, axis=1, keepdims=True)
    xh = xc * lax.rsqrt(var + 1e-5) * g_ref[...] + lb_ref[...]
    return jnp.where(xh > 0, xh, jnp.exp(jnp.minimum(xh, 0.0)) - 1.0)


def _tc_post_mm(a0, a1, a2, a3, deg_in, b, g, lb, deg_out, w_next):
    def body(a0_, a1_, a2_, a3_, di, b_, g_, lb_, do, w_, o0, o1, o2, o3):
        h = _postprocess((a0_, a1_, a2_, a3_), di, b_, g_, lb_)
        ns = lax.rsqrt(jnp.maximum(do[:, 0:1], 1.0))
        o = jnp.dot(h * ns, w_[...], preferred_element_type=jnp.float32)
        for p, oo in enumerate((o0, o1, o2, o3)):
            oo[...] = o[:, p * CW:(p + 1) * CW]

    chunk = pl.BlockSpec((_BN, CW), lambda i: (i, 0))
    vec = pl.BlockSpec((1, 128), lambda i: (0, 0))
    return pl.pallas_call(
        body,
        grid=(_GRID,),
        in_specs=[chunk] * 4 + [
            pl.BlockSpec((_BN, DW), lambda i: (i, 0)), vec, vec, vec,
            pl.BlockSpec((_BN, DW), lambda i: (i, 0)),
            pl.BlockSpec((128, 128), lambda i: (0, 0)),
        ],
        out_specs=[chunk] * 4,
        out_shape=[jax.ShapeDtypeStruct((N_PAD, CW), jnp.float32)] * 4,
    )(a0, a1, a2, a3, deg_in, b, g, lb, deg_out, w_next)


def _tc_post_final(a0, a1, a2, a3, deg_in, b, g, lb):
    def body(a0_, a1_, a2_, a3_, di, b_, g_, lb_, o):
        h = _postprocess((a0_, a1_, a2_, a3_), di, b_, g_, lb_)
        row = (pl.program_id(0) * _BN
               + lax.broadcasted_iota(jnp.int32, (_BN, 1), 0))
        o[...] = jnp.where(row < N_NODES, h, 0.0)

    chunk = pl.BlockSpec((_BN, CW), lambda i: (i, 0))
    vec = pl.BlockSpec((1, 128), lambda i: (0, 0))
    return pl.pallas_call(
        body,
        grid=(_GRID,),
        in_specs=[chunk] * 4 + [
            pl.BlockSpec((_BN, DW), lambda i: (i, 0)), vec, vec, vec,
        ],
        out_specs=pl.BlockSpec((_BN, 128), lambda i: (i, 0)),
        out_shape=jax.ShapeDtypeStruct((N_PAD, 128), jnp.float32),
    )(a0, a1, a2, a3, deg_in, b, g, lb)


def _tc_colsum(h):
    def body(h_ref, o_ref):
        @pl.when(pl.program_id(0) == 0)
        def _():
            o_ref[...] = jnp.zeros_like(o_ref)

        part = jnp.sum(h_ref[...], axis=0, keepdims=True)
        o_ref[...] += jnp.broadcast_to(part, (8, 128))

    return pl.pallas_call(
        body,
        grid=(_GRID,),
        in_specs=[pl.BlockSpec((_BN, 128), lambda i: (i, 0))],
        out_specs=pl.BlockSpec((8, 128), lambda i: (0, 0)),
        out_shape=jax.ShapeDtypeStruct((8, 128), jnp.float32),
    )(h)


def _tc_head(sn, sl, w0, b0, w1, b1):
    def body(sn_, sl_, w0_, b0_, w1_, b1_, o):
        hg = (sn_[0:1, :] + sl_[0:1, :]) * (1.0 / N_NODES)
        t = jnp.maximum(
            jnp.dot(hg, w0_[...], preferred_element_type=jnp.float32)
            + b0_[...], 0.0)
        r = jnp.dot(t, w1_[...], preferred_element_type=jnp.float32) + b1_[...]
        o[...] = jnp.broadcast_to(r, (8, 128))

    return pl.pallas_call(
        body,
        out_shape=jax.ShapeDtypeStruct((8, 128), jnp.float32),
    )(sn, sl, w0, b0, w1, b1)


def kernel(feat_n, feat_l, src_l2n, dst_l2n, src_n2l, dst_n2l, params):
    f32 = jnp.float32
    i32 = jnp.int32

    e = src_l2n.shape[0]
    e_pad = ((e + 4095) // 4096) * 4096

    def pad_idx(a):
        return jnp.concatenate(
            [a.astype(i32), jnp.full((e_pad - e,), TRASH, i32)])

    sl2n, dl2n = pad_idx(src_l2n), pad_idx(dst_l2n)
    sn2l, dn2l = pad_idx(src_n2l), pad_idx(dst_n2l)
    sl2n2, dl2n2 = sl2n.reshape(-1, EB), dl2n.reshape(-1, EB)
    sn2l2, dn2l2 = sn2l.reshape(-1, EB), dn2l.reshape(-1, EB)

    kn = feat_n.shape[1]
    kl = feat_l.shape[1]
    kn_p = ((kn + 127) // 128) * 128
    kl_p = ((kl + 127) // 128) * 128
    fn = jnp.pad(feat_n.astype(f32), ((0, N_PAD - feat_n.shape[0]),
                                      (0, kn_p - kn)))
    fl = jnp.pad(feat_l.astype(f32), ((0, N_PAD - feat_l.shape[0]),
                                      (0, kl_p - kl)))
    w0_l2n = jnp.pad(params["W0_l2n"].astype(f32), ((0, kl_p - kl), (0, 0)))
    w0_n2l = jnp.pad(params["W0_n2l"].astype(f32), ((0, kn_p - kn), (0, 0)))

    d0, d1, d2, d3 = _sc_degrees(sl2n, dl2n, sn2l, dn2l)

    vec = lambda v: v.astype(f32).reshape(1, -1)

    h_l2n = _tc_project(fl, w0_l2n, d0)
    h_n2l = _tc_project(fn, w0_n2l, d2)

    for i in range(3):
        a_n = _sc_segsum(*h_l2n, sl2n2, dl2n2)
        a_l = _sc_segsum(*h_n2l, sn2l2, dn2l2)
        b_n = vec(params["b%d_l2n" % i])
        b_l = vec(params["b%d_n2l" % i])
        g_n, lb_n = vec(params["ln%d_n_g" % i]), vec(params["ln%d_n_b" % i])
        g_l, lb_l = vec(params["ln%d_l_g" % i]), vec(params["ln%d_l_b" % i])
        if i < 2:
            h_n2l = _tc_post_mm(*a_n, d1, b_n, g_n, lb_n, d2,
                                params["W%d_n2l" % (i + 1)].astype(f32))
            h_l2n = _tc_post_mm(*a_l, d3, b_l, g_l, lb_l, d0,
                                params["W%d_l2n" % (i + 1)].astype(f32))
        else:
            hn = _tc_post_final(*a_n, d1, b_n, g_n, lb_n)
            hl = _tc_post_final(*a_l, d3, b_l, g_l, lb_l)

    sn = _tc_colsum(hn)
    sl = _tc_colsum(hl)
    w1 = jnp.pad(params["out_W"].astype(f32), ((0, 0), (0, 127)))
    b1 = jnp.pad(params["out_b"].astype(f32).reshape(1, 1),
                 ((0, 0), (0, 127)))
    res = _tc_head(sn, sl, params["fc0_W"].astype(f32), vec(params["fc0_b"]),
                   w1, b1)
    return res[0:1, 0:1]


# speedup vs baseline: 4.8400x; 1.2591x over previous
import functools

import jax
import jax.numpy as jnp
from jax import lax
from jax.experimental import pallas as pl
from jax.experimental.pallas import tpu as pltpu
from jax.experimental.pallas import tpu_sc as plsc

N_NODES = 50000
N_PAD = 50176
TRASH = 50100
CW = 32
DW = 16
ROWS_PER_TILE = N_PAD // 16
EB = 128

_MESH = dict(core_axis_name="c", subcore_axis_name="s",
             num_cores=2, num_subcores=16)


DSTRIPS = 4
DTQ = 784


def _sc_degrees(a0, a1, a2, a3):
    erows = a0.shape[0]
    rpt = erows // 16
    sb = rpt // DSTRIPS

    out_t = [jax.ShapeDtypeStruct((N_PAD, DW), jnp.float32)] * 4
    scratch = [
        pltpu.VMEM_SHARED((N_PAD, DW), jnp.float32),
        pltpu.VMEM_SHARED((N_PAD, DW), jnp.float32),
        pltpu.VMEM((2, 2, DSTRIPS, EB), jnp.int32),
        pltpu.VMEM((EB, DW), jnp.float32),
        pltpu.VMEM((DTQ, DW), jnp.float32),
        pltpu.SemaphoreType.DMA,
        pltpu.SemaphoreType.DMA,
    ]

    @functools.partial(
        pl.kernel,
        out_type=out_t,
        mesh=plsc.VectorSubcoreMesh(**_MESH),
        scratch_types=scratch,
        compiler_params=pltpu.CompilerParams(use_tc_tiling_on_sc=False),
    )
    def k(r0_, r1_, r2_, r3_, o0, o1, o2, o3, acc0, acc1, ibuf, ones, tmp,
          isem, ssem):
        c = lax.axis_index("c")
        s = lax.axis_index("s")
        arrs = (r0_, r1_, r2_, r3_)
        outs = (o0, o1, o2, o3)
        accs = (acc0, acc1)
        one16 = jnp.ones((DW,), jnp.float32)
        zero16 = jnp.zeros((DW,), jnp.float32)

        @pl.loop(0, EB)
        def _(i):
            ones[i, :] = one16

        @pl.loop(0, DTQ)
        def _(i):
            tmp[i, :] = zero16

        r0 = s * ROWS_PER_TILE
        for acc in accs:
            for q in range(ROWS_PER_TILE // DTQ):
                pltpu.sync_copy(tmp, acc.at[pl.ds(r0 + q * DTQ, DTQ)])
        plsc.subcore_barrier()

        base_r = s * rpt
        ones_b = ones.at[pl.ds(0, EB)]

        for cc in range(2):

            @pl.when(c == cc)
            def _():
                arr_a = arrs[2 * cc]
                arr_b = arrs[2 * cc + 1]

                def idx_dma(arr, j, g, slot, wait):
                    src = arr.at[pl.ds(base_r + DSTRIPS * g, DSTRIPS)]
                    if wait:
                        pltpu.make_async_copy(src, ibuf.at[j, slot],
                                              isem).wait()
                    else:
                        pltpu.async_copy(src, ibuf.at[j, slot], isem)

                idx_dma(arr_a, 0, 0, 0, wait=False)
                idx_dma(arr_b, 1, 0, 0, wait=False)

                @pl.loop(0, sb)
                def _(g):
                    slot = lax.rem(g, 2)
                    idx_dma(arr_a, 0, g, slot, wait=True)
                    idx_dma(arr_b, 1, g, slot, wait=True)
                    for j in range(DSTRIPS):
                        pltpu.async_copy(ones_b, acc0.at[ibuf.at[0, slot, j]],
                                         ssem, add=True)
                        pltpu.async_copy(ones_b, acc1.at[ibuf.at[1, slot, j]],
                                         ssem, add=True)

                    @pl.when(g + 1 < sb)
                    def _():
                        ns = lax.rem(g + 1, 2)
                        idx_dma(arr_a, 0, g + 1, ns, wait=False)
                        idx_dma(arr_b, 1, g + 1, ns, wait=False)

                    for _j in range(2 * DSTRIPS):
                        pltpu.make_async_copy(o0.at[pl.ds(0, EB)], ones,
                                              ssem).wait()

        plsc.subcore_barrier()
        for cc in range(2):

            @pl.when(c == cc)
            def _():
                for j in range(2):
                    acc = accs[j]
                    out = outs[2 * cc + j]
                    for q in range(ROWS_PER_TILE // DTQ):
                        rr = r0 + q * DTQ
                        pltpu.sync_copy(acc.at[pl.ds(rr, DTQ)], tmp)
                        pltpu.sync_copy(tmp, out.at[pl.ds(rr, DTQ)])

    return k(a0, a1, a2, a3)


SBE = 256
STRIPS = SBE // EB
TQ = 196


def _sc_segsum(h0, h1, h2, h3, src2, dst2):
    erows = src2.shape[0]
    rpt = erows // 16
    sb = rpt // STRIPS
    out_t = [jax.ShapeDtypeStruct((N_PAD, CW), jnp.float32)] * 4
    scratch = [
        pltpu.VMEM_SHARED((N_PAD, CW), jnp.float32),
        pltpu.VMEM((3, STRIPS, EB), jnp.int32),
        pltpu.VMEM((3, STRIPS, EB), jnp.int32),
        pltpu.VMEM((2, SBE, CW), jnp.float32),
        pltpu.VMEM((TQ, CW), jnp.float32),
        pltpu.SemaphoreType.DMA,
        pltpu.SemaphoreType.DMA,
        pltpu.SemaphoreType.DMA,
    ]

    @functools.partial(
        pl.kernel,
        out_type=out_t,
        mesh=plsc.VectorSubcoreMesh(**_MESH),
        scratch_types=scratch,
        compiler_params=pltpu.CompilerParams(use_tc_tiling_on_sc=False),
    )
    def k(h0_, h1_, h2_, h3_, se, de, o0, o1, o2, o3, acc, sbuf, dbuf, rows,
          tmp, isem, gsem, ssem):
        c = lax.axis_index("c")
        s = lax.axis_index("s")
        hs = (h0_, h1_, h2_, h3_)
        outs = (o0, o1, o2, o3)
        z16 = jnp.zeros((16,), jnp.float32)
        r0 = s * ROWS_PER_TILE
        base_r = s * rpt

        def idx_dma(g, slot, wait):
            srcs = se.at[pl.ds(base_r + STRIPS * g, STRIPS)]
            dsts = de.at[pl.ds(base_r + STRIPS * g, STRIPS)]
            if wait:
                pltpu.make_async_copy(srcs, sbuf.at[slot], isem).wait()
                pltpu.make_async_copy(dsts, dbuf.at[slot], isem).wait()
            else:
                pltpu.async_copy(srcs, sbuf.at[slot], isem)
                pltpu.async_copy(dsts, dbuf.at[slot], isem)

        for cc in range(2):

            @pl.when(c == cc)
            def _():
                for p in range(2):
                    ch = 2 * cc + p
                    h = hs[ch]
                    drain_src = h.at[pl.ds(0, SBE)]

                    @pl.loop(0, TQ)
                    def _(i):
                        tmp[i, pl.ds(0, 16)] = z16
                        tmp[i, pl.ds(16, 16)] = z16

                    for q in range(16):
                        pltpu.sync_copy(tmp, acc.at[pl.ds(r0 + q * TQ, TQ)])
                    plsc.subcore_barrier()

                    idx_dma(0, 0, wait=False)

                    @pl.loop(0, sb)
                    def _(g):
                        slot = lax.rem(g, 3)
                        rb = lax.rem(g, 2)

                        idx_dma(g, slot, wait=True)
                        for j in range(STRIPS):
                            pltpu.async_copy(
                                h.at[sbuf.at[slot, j]],
                                rows.at[rb, pl.ds(j * EB, EB)], gsem)

                        @pl.when(g + 1 < sb)
                        def _():
                            idx_dma(g + 1, lax.rem(g + 1, 3), wait=False)

                        @pl.when(g >= 1)
                        def _():
                            pltpu.make_async_copy(drain_src,
                                                  rows.at[1 - rb],
                                                  ssem).wait()

                        pltpu.make_async_copy(drain_src, rows.at[rb],
                                              gsem).wait()
                        for j in range(STRIPS):
                            pltpu.async_copy(
                                rows.at[rb, pl.ds(j * EB, EB)],
                                acc.at[dbuf.at[slot, j]], ssem, add=True)

                    pltpu.make_async_copy(drain_src, rows.at[0],
                                          ssem).wait()
                    plsc.subcore_barrier()
                    for q in range(16):
                        rr = r0 + q * TQ
                        pltpu.sync_copy(acc.at[pl.ds(rr, TQ)], tmp)
                        pltpu.sync_copy(tmp, outs[ch].at[pl.ds(rr, TQ)])
                    plsc.subcore_barrier()

    return k(h0, h1, h2, h3, src2, dst2)


_BN = 512
_GRID = N_PAD // _BN


def _tc_project(x, w, deg):
    kdim = x.shape[1]

    def body(x_ref, w_ref, d_ref, o0, o1, o2, o3):
        ns = lax.rsqrt(jnp.maximum(d_ref[:, 0:1], 1.0))
        h = jnp.dot(x_ref[...], w_ref[...],
                    preferred_element_type=jnp.float32) * ns
        for p, o in enumerate((o0, o1, o2, o3)):
            o[...] = h[:, p * CW:(p + 1) * CW]

    return pl.pallas_call(
        body,
        grid=(_GRID,),
        in_specs=[
            pl.BlockSpec((_BN, kdim), lambda i: (i, 0)),
            pl.BlockSpec((kdim, 128), lambda i: (0, 0)),
            pl.BlockSpec((_BN, DW), lambda i: (i, 0)),
        ],
        out_specs=[pl.BlockSpec((_BN, CW), lambda i: (i, 0))] * 4,
        out_shape=[jax.ShapeDtypeStruct((N_PAD, CW), jnp.float32)] * 4,
    )(x, w, deg)


def _postprocess(a_refs, di_ref, b_ref, g_ref, lb_ref):
    a = jnp.concatenate([r[...] for r in a_refs], axis=1)
    nd = lax.rsqrt(jnp.maximum(di_ref[:, 0:1], 1.0))
    x = a * nd + b_ref[...]
    mu = jnp.mean(x, axis=1, keepdims=True)
    xc = x - mu
    var = jnp.mean(xc * xc, axis=1, keepdims=True)
    xh = xc * lax.rsqrt(var + 1e-5) * g_ref[...] + lb_ref[...]
    return jnp.where(xh > 0, xh, jnp.exp(jnp.minimum(xh, 0.0)) - 1.0)


def _tc_post_mm(a0, a1, a2, a3, deg_in, b, g, lb, deg_out, w_next):
    def body(a0_, a1_, a2_, a3_, di, b_, g_, lb_, do, w_, o0, o1, o2, o3):
        h = _postprocess((a0_, a1_, a2_, a3_), di, b_, g_, lb_)
        ns = lax.rsqrt(jnp.maximum(do[:, 0:1], 1.0))
        o = jnp.dot(h * ns, w_[...], preferred_element_type=jnp.float32)
        for p, oo in enumerate((o0, o1, o2, o3)):
            oo[...] = o[:, p * CW:(p + 1) * CW]

    chunk = pl.BlockSpec((_BN, CW), lambda i: (i, 0))
    vec = pl.BlockSpec((1, 128), lambda i: (0, 0))
    return pl.pallas_call(
        body,
        grid=(_GRID,),
        in_specs=[chunk] * 4 + [
            pl.BlockSpec((_BN, DW), lambda i: (i, 0)), vec, vec, vec,
            pl.BlockSpec((_BN, DW), lambda i: (i, 0)),
            pl.BlockSpec((128, 128), lambda i: (0, 0)),
        ],
        out_specs=[chunk] * 4,
        out_shape=[jax.ShapeDtypeStruct((N_PAD, CW), jnp.float32)] * 4,
    )(a0, a1, a2, a3, deg_in, b, g, lb, deg_out, w_next)


def _tc_post_final(a0, a1, a2, a3, deg_in, b, g, lb):
    def body(a0_, a1_, a2_, a3_, di, b_, g_, lb_, o):
        h = _postprocess((a0_, a1_, a2_, a3_), di, b_, g_, lb_)
        row = (pl.program_id(0) * _BN
               + lax.broadcasted_iota(jnp.int32, (_BN, 1), 0))
        o[...] = jnp.where(row < N_NODES, h, 0.0)

    chunk = pl.BlockSpec((_BN, CW), lambda i: (i, 0))
    vec = pl.BlockSpec((1, 128), lambda i: (0, 0))
    return pl.pallas_call(
        body,
        grid=(_GRID,),
        in_specs=[chunk] * 4 + [
            pl.BlockSpec((_BN, DW), lambda i: (i, 0)), vec, vec, vec,
        ],
        out_specs=pl.BlockSpec((_BN, 128), lambda i: (i, 0)),
        out_shape=jax.ShapeDtypeStruct((N_PAD, 128), jnp.float32),
    )(a0, a1, a2, a3, deg_in, b, g, lb)


def _tc_colsum(h):
    def body(h_ref, o_ref):
        @pl.when(pl.program_id(0) == 0)
        def _():
            o_ref[...] = jnp.zeros_like(o_ref)

        part = jnp.sum(h_ref[...], axis=0, keepdims=True)
        o_ref[...] += jnp.broadcast_to(part, (8, 128))

    return pl.pallas_call(
        body,
        grid=(_GRID,),
        in_specs=[pl.BlockSpec((_BN, 128), lambda i: (i, 0))],
        out_specs=pl.BlockSpec((8, 128), lambda i: (0, 0)),
        out_shape=jax.ShapeDtypeStruct((8, 128), jnp.float32),
    )(h)


def _tc_head(sn, sl, w0, b0, w1, b1):
    def body(sn_, sl_, w0_, b0_, w1_, b1_, o):
        hg = (sn_[0:1, :] + sl_[0:1, :]) * (1.0 / N_NODES)
        t = jnp.maximum(
            jnp.dot(hg, w0_[...], preferred_element_type=jnp.float32)
            + b0_[...], 0.0)
        r = jnp.dot(t, w1_[...], preferred_element_type=jnp.float32) + b1_[...]
        o[...] = jnp.broadcast_to(r, (8, 128))

    return pl.pallas_call(
        body,
        out_shape=jax.ShapeDtypeStruct((8, 128), jnp.float32),
    )(sn, sl, w0, b0, w1, b1)


def kernel(feat_n, feat_l, src_l2n, dst_l2n, src_n2l, dst_n2l, params):
    f32 = jnp.float32
    i32 = jnp.int32

    e = src_l2n.shape[0]
    e_pad = ((e + 4095) // 4096) * 4096

    def pad_idx(a):
        return jnp.concatenate(
            [a.astype(i32), jnp.full((e_pad - e,), TRASH, i32)])

    sl2n, dl2n = pad_idx(src_l2n), pad_idx(dst_l2n)
    sn2l, dn2l = pad_idx(src_n2l), pad_idx(dst_n2l)
    sl2n2, dl2n2 = sl2n.reshape(-1, EB), dl2n.reshape(-1, EB)
    sn2l2, dn2l2 = sn2l.reshape(-1, EB), dn2l.reshape(-1, EB)

    kn = feat_n.shape[1]
    kl = feat_l.shape[1]
    kn_p = ((kn + 127) // 128) * 128
    kl_p = ((kl + 127) // 128) * 128
    fn = jnp.pad(feat_n.astype(f32), ((0, N_PAD - feat_n.shape[0]),
                                      (0, kn_p - kn)))
    fl = jnp.pad(feat_l.astype(f32), ((0, N_PAD - feat_l.shape[0]),
                                      (0, kl_p - kl)))
    w0_l2n = jnp.pad(params["W0_l2n"].astype(f32), ((0, kl_p - kl), (0, 0)))
    w0_n2l = jnp.pad(params["W0_n2l"].astype(f32), ((0, kn_p - kn), (0, 0)))

    d0, d1, d2, d3 = _sc_degrees(sl2n2, dl2n2, sn2l2, dn2l2)

    vec = lambda v: v.astype(f32).reshape(1, -1)

    h_l2n = _tc_project(fl, w0_l2n, d0)
    h_n2l = _tc_project(fn, w0_n2l, d2)

    for i in range(3):
        a_n = _sc_segsum(*h_l2n, sl2n2, dl2n2)
        a_l = _sc_segsum(*h_n2l, sn2l2, dn2l2)
        b_n = vec(params["b%d_l2n" % i])
        b_l = vec(params["b%d_n2l" % i])
        g_n, lb_n = vec(params["ln%d_n_g" % i]), vec(params["ln%d_n_b" % i])
        g_l, lb_l = vec(params["ln%d_l_g" % i]), vec(params["ln%d_l_b" % i])
        if i < 2:
            h_n2l = _tc_post_mm(*a_n, d1, b_n, g_n, lb_n, d2,
                                params["W%d_n2l" % (i + 1)].astype(f32))
            h_l2n = _tc_post_mm(*a_l, d3, b_l, g_l, lb_l, d0,
                                params["W%d_l2n" % (i + 1)].astype(f32))
        else:
            hn = _tc_post_final(*a_n, d1, b_n, g_n, lb_n)
            hl = _tc_post_final(*a_l, d3, b_l, g_l, lb_l)

    sn = _tc_colsum(hn)
    sl = _tc_colsum(hl)
    w1 = jnp.pad(params["out_W"].astype(f32), ((0, 0), (0, 127)))
    b1 = jnp.pad(params["out_b"].astype(f32).reshape(1, 1),
                 ((0, 0), (0, 127)))
    res = _tc_head(sn, sl, params["fc0_W"].astype(f32), vec(params["fc0_b"]),
                   w1, b1)
    return res[0:1, 0:1]
